# baseline (speedup 1.0000x reference)
import functools

import jax
import jax.numpy as jnp
from jax import lax
from jax.experimental import pallas as pl
from jax.experimental.pallas import tpu as pltpu
from jax.experimental.pallas import tpu_sc as plsc

_N = 10000
_E = 320000
_D = 128
_BLOCKS = 5
_G = 64
_C = 10

_NW = 32
_TILES = 16
_CHUNK = 128
_CPW0 = 128
_CPW1 = 32
_EPAD = _TILES * (_CPW0 + _CPW1) * _CHUNK
_NPAD = 10240
_RPT = _NPAD // _TILES
_ZROWS = 16
_STG = 16

_mesh = plsc.VectorSubcoreMesh(core_axis_name="c", subcore_axis_name="s")


@functools.partial(
    pl.kernel,
    mesh=_mesh,
    out_type=jax.ShapeDtypeStruct((2 * _NPAD, _D), jnp.float32),
    scratch_types=[
        pltpu.VMEM((_STG, _CHUNK), jnp.int32),
        pltpu.VMEM((_STG, _CHUNK), jnp.int32),
        pltpu.VMEM((_CHUNK, _D), jnp.float32),
        pltpu.VMEM((_CHUNK, _D), jnp.float32),
        pltpu.VMEM((_ZROWS, _D), jnp.float32),
        pltpu.VMEM_SHARED((_NPAD, _D), jnp.float32),
        pltpu.SemaphoreType.DMA,
        pltpu.SemaphoreType.DMA,
    ],
)
def _sc_agg(h_hbm, src_hbm, dst_hbm, out_hbm, sidx, didx, rows_a, rows_b,
            zbuf, acc, sem_a, sem_b):
    cid = lax.axis_index("c")
    sid = lax.axis_index("s")
    base = jnp.where(cid == 0, sid * _CPW0, _TILES * _CPW0 + sid * _CPW1)
    nstg = jnp.where(cid == 0, _CPW0 // _STG, _CPW1 // _STG)

    def _zstore(i, carry):
        r = i // 8
        col = (i % 8) * 16
        zbuf[r, pl.ds(col, 16)] = jnp.zeros((16,), jnp.float32)
        return carry

    lax.fori_loop(0, _ZROWS * 8, _zstore, 0)

    def _zcopy(k, carry):
        pltpu.sync_copy(zbuf, acc.at[pl.ds(sid * _RPT + k * _ZROWS, _ZROWS)])
        return carry

    lax.fori_loop(0, _RPT // _ZROWS, _zcopy, 0)
    plsc.subcore_barrier()

    def _stage(q, carry):
        row0 = pl.multiple_of(base + q * _STG, 8)
        pltpu.sync_copy(src_hbm.at[pl.ds(row0, _STG)], sidx)
        pltpu.sync_copy(dst_hbm.at[pl.ds(row0, _STG)], didx)
        pltpu.async_copy(h_hbm.at[sidx.at[0]], rows_a, sem_a)

        def _step(t, c2):
            c0 = t * 2
            c1 = c0 + 1
            pltpu.async_copy(h_hbm.at[sidx.at[c1]], rows_b, sem_b)
            pltpu.make_async_copy(h_hbm.at[sidx.at[0]], rows_a, sem_a).wait()
            pltpu.sync_copy(rows_a, acc.at[didx.at[c0]], add=True)

            @pl.when(c1 + 1 < _STG)
            def _():
                pltpu.async_copy(h_hbm.at[sidx.at[c1 + 1]], rows_a, sem_a)

            pltpu.make_async_copy(h_hbm.at[sidx.at[0]], rows_b, sem_b).wait()
            pltpu.sync_copy(rows_b, acc.at[didx.at[c1]], add=True)
            return c2

        lax.fori_loop(0, _STG // 2, _step, 0)
        return carry

    lax.fori_loop(0, nstg, _stage, 0)
    plsc.subcore_barrier()

    pltpu.sync_copy(
        acc.at[pl.ds(sid * _RPT, _RPT)],
        out_hbm.at[pl.ds(cid * _NPAD + sid * _RPT, _RPT)],
    )


def _dense_body(h_ref, agg_ref, w1_ref, b1_ref, g1_ref, be1_ref,
                w2_ref, b2_ref, g2_ref, be2_ref, batch_ref,
                hout_ref, feat_ref):
    hin = h_ref[...] + agg_ref[0:_N, :] + agg_ref[_NPAD:_NPAD + _N, :]
    y = jnp.dot(hin, w1_ref[...], preferred_element_type=jnp.float32,
                precision=lax.Precision.HIGHEST) + b1_ref[...]
    y = jnp.maximum(y, 0.0)
    m = jnp.mean(y, axis=0, keepdims=True)
    v = jnp.mean((y - m) ** 2, axis=0, keepdims=True)
    y = (y - m) * lax.rsqrt(v + 1e-5) * g1_ref[...] + be1_ref[...]
    z = jnp.dot(y, w2_ref[...], preferred_element_type=jnp.float32,
                precision=lax.Precision.HIGHEST) + b2_ref[...]
    z = jnp.maximum(z, 0.0)
    m2 = jnp.mean(z, axis=0, keepdims=True)
    v2 = jnp.mean((z - m2) ** 2, axis=0, keepdims=True)
    z = (z - m2) * lax.rsqrt(v2 + 1e-5) * g2_ref[...] + be2_ref[...]
    hout_ref[...] = z
    onehot = (lax.broadcasted_iota(jnp.int32, (_G, _N), 0)
              == batch_ref[...]).astype(jnp.float32)
    feat_ref[...] = jnp.dot(onehot, z, preferred_element_type=jnp.float32,
                            precision=lax.Precision.HIGHEST)


_dense = pl.pallas_call(
    _dense_body,
    out_shape=(
        jax.ShapeDtypeStruct((_N, _D), jnp.float32),
        jax.ShapeDtypeStruct((_G, _D), jnp.float32),
    ),
)


def _cls_body(f_ref, gc_ref, bcn_ref, wc1_ref, bc1_ref, wc2_ref, bc2_ref,
              out_ref):
    f = f_ref[...]
    m = jnp.mean(f, axis=0, keepdims=True)
    v = jnp.mean((f - m) ** 2, axis=0, keepdims=True)
    f = (f - m) * lax.rsqrt(v + 1e-5) * gc_ref[...] + bcn_ref[...]
    z = jnp.dot(f, wc1_ref[...], preferred_element_type=jnp.float32,
                precision=lax.Precision.HIGHEST) + bc1_ref[...]
    z = jnp.maximum(z, 0.0)
    z = jnp.dot(z, wc2_ref[...], preferred_element_type=jnp.float32,
                precision=lax.Precision.HIGHEST) + bc2_ref[...]
    zmax = jnp.max(z, axis=-1, keepdims=True)
    lse = zmax + jnp.log(jnp.sum(jnp.exp(z - zmax), axis=-1, keepdims=True))
    out_ref[...] = z - lse


_classifier = pl.pallas_call(
    _cls_body,
    out_shape=jax.ShapeDtypeStruct((_G, _C), jnp.float32),
)


def kernel(x, edge_index, batch, W1, b1, g1, be1, W2, b2, g2, be2,
           gc, bcn, Wc1, bc1, Wc2, bc2):
    src = edge_index[0]
    dst = edge_index[1]
    pad = _EPAD - _E
    src_p = jnp.concatenate([src, jnp.zeros((pad,), jnp.int32)]).reshape(
        _EPAD // _CHUNK, _CHUNK)
    dst_p = jnp.concatenate([dst, jnp.full((pad,), _N + 16, jnp.int32)]).reshape(
        _EPAD // _CHUNK, _CHUNK)
    batch2d = batch.reshape(1, _N)

    h = x
    feats = []
    for i in range(_BLOCKS):
        agg = _sc_agg(h, src_p, dst_p)
        h, f = _dense(h, agg, W1[i], b1[i].reshape(1, _D), g1[i].reshape(1, _D),
                      be1[i].reshape(1, _D), W2[i], b2[i].reshape(1, _D),
                      g2[i].reshape(1, _D), be2[i].reshape(1, _D), batch2d)
        feats.append(f)
    fcat = jnp.concatenate(feats, axis=1)
    return _classifier(fcat, gc.reshape(1, _BLOCKS * _D),
                       bcn.reshape(1, _BLOCKS * _D), Wc1,
                       bc1.reshape(1, _D), Wc2, bc2.reshape(1, _C))


# speedup vs baseline: 1.2044x; 1.2044x over previous
import functools

import jax
import jax.numpy as jnp
from jax import lax
from jax.experimental import pallas as pl
from jax.experimental.pallas import tpu as pltpu
from jax.experimental.pallas import tpu_sc as plsc

_N = 10000
_E = 320000
_D = 128
_BLOCKS = 5
_G = 64
_C = 10

_NW = 32
_TILES = 16
_CHUNK = 128
_CPW = 80
_EPAD = _NW * _CPW * _CHUNK
_NPAD = 10240
_RPT = _NPAD // _TILES
_ZROWS = 16
_DH = _D // 2
_CPT = _EPAD // _CHUNK // _TILES
_QCPT = 40
_NBUF = 4

_mesh = plsc.VectorSubcoreMesh(core_axis_name="c", subcore_axis_name="s")


@functools.partial(
    pl.kernel,
    mesh=_mesh,
    compiler_params=pltpu.CompilerParams(use_tc_tiling_on_sc=False),
    out_type=(jax.ShapeDtypeStruct((_NPAD, _DH), jnp.float32),
              jax.ShapeDtypeStruct((_NPAD, _DH), jnp.float32)),
    scratch_types=[
        pltpu.VMEM((_QCPT, _CHUNK), jnp.int32),
        pltpu.VMEM((_QCPT, _CHUNK), jnp.int32),
        pltpu.VMEM((_NBUF, _CHUNK, _DH), jnp.float32),
        pltpu.VMEM((_ZROWS, _DH), jnp.float32),
        pltpu.VMEM_SHARED((_NPAD, _DH), jnp.float32),
        pltpu.SemaphoreType.DMA,
        pltpu.SemaphoreType.DMA,
        pltpu.SemaphoreType.DMA,
        pltpu.SemaphoreType.DMA,
    ],
)
def _sc_agg(hl_hbm, hr_hbm, src_hbm, dst_hbm, outl_hbm, outr_hbm,
            sidx, didx, rows, zbuf, acc, sem0, sem1, sem2, sem3):
    cid = lax.axis_index("c")
    sid = lax.axis_index("s")
    sems = (sem0, sem1, sem2, sem3)

    def _zstore(i, carry):
        r = i // 4
        col = (i % 4) * 16
        zbuf[r, pl.ds(col, 16)] = jnp.zeros((16,), jnp.float32)
        return carry

    lax.fori_loop(0, _ZROWS * 4, _zstore, 0)

    def _zcopy(k, carry):
        pltpu.sync_copy(zbuf, acc.at[pl.ds(sid * _RPT + k * _ZROWS, _ZROWS)])
        return carry

    lax.fori_loop(0, _RPT // _ZROWS, _zcopy, 0)
    plsc.subcore_barrier()

    def _issue(c, b):
        @pl.when(cid == 0)
        def _():
            pltpu.async_copy(hl_hbm.at[sidx.at[c]], rows.at[b], sems[b])

        @pl.when(cid == 1)
        def _():
            pltpu.async_copy(hr_hbm.at[sidx.at[c]], rows.at[b], sems[b])

    def _drain(c, b):
        pltpu.make_async_copy(hl_hbm.at[pl.ds(0, _CHUNK)], rows.at[b],
                              sems[b]).wait()
        pltpu.sync_copy(rows.at[b], acc.at[didx.at[c]], add=True)

    def _stage(q, carry):
        row0 = pl.multiple_of(sid * _CPT + q * _QCPT, 8)
        pltpu.sync_copy(src_hbm.at[pl.ds(row0, _QCPT)], sidx)
        pltpu.sync_copy(dst_hbm.at[pl.ds(row0, _QCPT)], didx)
        for b in range(_NBUF):
            _issue(b, b)

        def _grp(t, c2):
            for b in range(_NBUF):
                c = t * _NBUF + b
                _drain(c, b)
                _issue(c + _NBUF, b)
            return c2

        lax.fori_loop(0, _QCPT // _NBUF - 1, _grp, 0)
        for b in range(_NBUF):
            _drain(_QCPT - _NBUF + b, b)
        return carry

    lax.fori_loop(0, _CPT // _QCPT, _stage, 0)
    plsc.subcore_barrier()

    @pl.when(cid == 0)
    def _():
        pltpu.sync_copy(acc.at[pl.ds(sid * _RPT, _RPT)],
                        outl_hbm.at[pl.ds(sid * _RPT, _RPT)])

    @pl.when(cid == 1)
    def _():
        pltpu.sync_copy(acc.at[pl.ds(sid * _RPT, _RPT)],
                        outr_hbm.at[pl.ds(sid * _RPT, _RPT)])


def _dense_body(hl_ref, hr_ref, aggl_ref, aggr_ref, w1_ref, b1_ref, g1_ref,
                be1_ref, w2_ref, b2_ref, g2_ref, be2_ref, batch_ref,
                houtl_ref, houtr_ref, feat_ref):
    hin = jnp.concatenate(
        [hl_ref[...] + aggl_ref[0:_N, :], hr_ref[...] + aggr_ref[0:_N, :]],
        axis=1)
    y = jnp.dot(hin, w1_ref[...], preferred_element_type=jnp.float32,
                precision=lax.Precision.HIGHEST) + b1_ref[...]
    y = jnp.maximum(y, 0.0)
    m = jnp.mean(y, axis=0, keepdims=True)
    v = jnp.mean((y - m) ** 2, axis=0, keepdims=True)
    y = (y - m) * lax.rsqrt(v + 1e-5) * g1_ref[...] + be1_ref[...]
    z = jnp.dot(y, w2_ref[...], preferred_element_type=jnp.float32,
                precision=lax.Precision.HIGHEST) + b2_ref[...]
    z = jnp.maximum(z, 0.0)
    m2 = jnp.mean(z, axis=0, keepdims=True)
    v2 = jnp.mean((z - m2) ** 2, axis=0, keepdims=True)
    z = (z - m2) * lax.rsqrt(v2 + 1e-5) * g2_ref[...] + be2_ref[...]
    houtl_ref[...] = z[:, 0:_DH]
    houtr_ref[...] = z[:, _DH:_D]
    onehot = (lax.broadcasted_iota(jnp.int32, (_G, _N), 0)
              == batch_ref[...]).astype(jnp.float32)
    feat_ref[...] = jnp.dot(onehot, z, preferred_element_type=jnp.float32,
                            precision=lax.Precision.HIGHEST)


_dense = pl.pallas_call(
    _dense_body,
    out_shape=(
        jax.ShapeDtypeStruct((_N, _DH), jnp.float32),
        jax.ShapeDtypeStruct((_N, _DH), jnp.float32),
        jax.ShapeDtypeStruct((_G, _D), jnp.float32),
    ),
)


def _cls_body(f_ref, gc_ref, bcn_ref, wc1_ref, bc1_ref, wc2_ref, bc2_ref,
              out_ref):
    f = f_ref[...]
    m = jnp.mean(f, axis=0, keepdims=True)
    v = jnp.mean((f - m) ** 2, axis=0, keepdims=True)
    f = (f - m) * lax.rsqrt(v + 1e-5) * gc_ref[...] + bcn_ref[...]
    z = jnp.dot(f, wc1_ref[...], preferred_element_type=jnp.float32,
                precision=lax.Precision.HIGHEST) + bc1_ref[...]
    z = jnp.maximum(z, 0.0)
    z = jnp.dot(z, wc2_ref[...], preferred_element_type=jnp.float32,
                precision=lax.Precision.HIGHEST) + bc2_ref[...]
    zmax = jnp.max(z, axis=-1, keepdims=True)
    lse = zmax + jnp.log(jnp.sum(jnp.exp(z - zmax), axis=-1, keepdims=True))
    out_ref[...] = z - lse


_classifier = pl.pallas_call(
    _cls_body,
    out_shape=jax.ShapeDtypeStruct((_G, _C), jnp.float32),
)


def kernel(x, edge_index, batch, W1, b1, g1, be1, W2, b2, g2, be2,
           gc, bcn, Wc1, bc1, Wc2, bc2):
    src = edge_index[0]
    dst = edge_index[1]
    pad = _EPAD - _E
    src_p = jnp.concatenate([src, jnp.zeros((pad,), jnp.int32)]).reshape(
        _EPAD // _CHUNK, _CHUNK)
    dst_p = jnp.concatenate([dst, jnp.full((pad,), _N + 16, jnp.int32)]).reshape(
        _EPAD // _CHUNK, _CHUNK)
    batch2d = batch.reshape(1, _N)

    hl = x[:, 0:_DH]
    hr = x[:, _DH:_D]
    feats = []
    for i in range(_BLOCKS):
        aggl, aggr = _sc_agg(hl, hr, src_p, dst_p)
        hl, hr, f = _dense(hl, hr, aggl, aggr, W1[i], b1[i].reshape(1, _D),
                           g1[i].reshape(1, _D), be1[i].reshape(1, _D), W2[i],
                           b2[i].reshape(1, _D), g2[i].reshape(1, _D),
                           be2[i].reshape(1, _D), batch2d)
        feats.append(f)
    fcat = jnp.concatenate(feats, axis=1)
    return _classifier(fcat, gc.reshape(1, _BLOCKS * _D),
                       bcn.reshape(1, _BLOCKS * _D), Wc1,
                       bc1.reshape(1, _D), Wc2, bc2.reshape(1, _C))
